# baseline (speedup 1.0000x reference)
import functools

import jax
import jax.numpy as jnp
from jax import lax
from jax.experimental import pallas as pl
from jax.experimental.pallas import tpu as pltpu
from jax.experimental.pallas import tpu_sc as plsc

DIM = 768
GRID_DIM = 24.0
SPAD = 592
NV = DIM // 16


def _score_body(ids_ref, x_ref, wq_ref, bq_ref, wk_ref, bk_ref, wv_ref,
                bv_ref, avg_ref, std_ref, eps_ref, v_ref, w_ref, idx_ref):
    S = x_ref.shape[1]
    xb = x_ref[0]
    q = jnp.dot(xb, wq_ref[...], preferred_element_type=jnp.float32) + bq_ref[...]
    k = jnp.dot(xb, wk_ref[...], preferred_element_type=jnp.float32) + bk_ref[...]
    v = jnp.dot(xb, wv_ref[...], preferred_element_type=jnp.float32) + bv_ref[...]
    v_ref[0] = v
    AT = jax.lax.dot_general(k, q, (((1,), (1,)), ((), ())),
                             preferred_element_type=jnp.float32)

    key_x = (eps_ref[0, :, 0:1] - avg_ref[0, 0:1, :]) / std_ref[0, 0:1, :]
    key_y = (eps_ref[0, :, 1:2] - avg_ref[0, 1:2, :]) / std_ref[0, 1:2, :]
    kx1, kx2 = jnp.ceil(key_x), jnp.floor(key_x)
    ky1, ky2 = jnp.ceil(key_y), jnp.floor(key_y)
    zero = jnp.zeros((1, 1), jnp.int32)
    idxs = []
    for fy, fx in ((ky1, kx1), (ky1, kx2), (ky2, kx1), (ky2, kx2)):
        ij = (GRID_DIM * fy + fx).astype(jnp.int32) % S
        idxs.append(jnp.concatenate([zero, ij], axis=1))

    rows = jax.lax.broadcasted_iota(jnp.int32, (S, S), 0)
    ats = []
    for j in range(4):
        mj = (rows == idxs[j]).astype(jnp.float32)
        ats.append(jnp.sum(AT * mj, axis=0, keepdims=True))
    at = jnp.concatenate(ats, axis=0)
    m = jnp.max(at, axis=0, keepdims=True)
    e = jnp.exp(at - m)
    w = e / jnp.sum(e, axis=0, keepdims=True)
    wpad = jnp.zeros((4, SPAD - S), jnp.float32)
    w_ref[0] = jnp.concatenate([w, wpad], axis=1)
    ipad = jnp.zeros((4, SPAD - S), jnp.int32)
    idx_ref[0] = jnp.concatenate(
        [jnp.concatenate(idxs, axis=0), ipad], axis=1)


def _make_sc_combine(B, S, D):
    n_chunks = SPAD // 16
    mesh = plsc.VectorSubcoreMesh(core_axis_name="c", subcore_axis_name="s")

    @functools.partial(
        pl.kernel, mesh=mesh,
        out_type=jax.ShapeDtypeStruct((B, S, D), jnp.float32),
        scratch_types=[
            pltpu.VMEM((4 * SPAD,), jnp.int32),
            pltpu.VMEM((4 * SPAD * 16,), jnp.float32),
            pltpu.VMEM((64,), jnp.int32),
            pltpu.VMEM((64, DIM), jnp.float32),
            pltpu.VMEM((16, DIM), jnp.float32),
            pltpu.SemaphoreType.DMA,
        ],
    )
    def sc_combine(vflat_hbm, idx_hbm, w_hbm, out_hbm,
                   idx_all, w_all, idx_v, rows_v, acc_v, sem):
        b = lax.axis_index("s") * 2 + lax.axis_index("c")
        pltpu.sync_copy(idx_hbm.at[b], idx_all)
        pltpu.sync_copy(w_hbm.at[b], w_all)
        base = b * S

        def chunk(g, _):
            s0 = g * 16
            for j in range(4):
                idx_v[pl.ds(j * 16, 16)] = idx_all[pl.ds(j * SPAD + s0, 16)] + base
            pltpu.async_copy(vflat_hbm.at[idx_v], rows_v, sem).wait()

            def pos(sl, _):
                s = s0 + sl
                ws = [w_all[pl.ds((j * SPAD + s) * 16, 16)] for j in range(4)]
                for i in range(NV):
                    acc = ws[0] * rows_v[sl, pl.ds(i * 16, 16)]
                    for j in range(1, 4):
                        acc = acc + ws[j] * rows_v[j * 16 + sl, pl.ds(i * 16, 16)]
                    acc_v[sl, pl.ds(i * 16, 16)] = acc
                return _

            lax.fori_loop(0, 16, pos, None)

            @pl.when(g == 0)
            def _():
                ones = jnp.full((16,), 1.0, jnp.float32)
                for i in range(NV):
                    acc_v[0, pl.ds(i * 16, 16)] = ones

            @pl.when(g < n_chunks - 1)
            def _():
                pltpu.sync_copy(acc_v, out_hbm.at[b, pl.ds(s0, 16)])

            @pl.when(g == n_chunks - 1)
            def _():
                pltpu.sync_copy(acc_v.at[pl.ds(0, 1)],
                                out_hbm.at[b, pl.ds(s0, 1)])
            return _

        lax.fori_loop(0, n_chunks, chunk, None)

    return sc_combine


def kernel(x, img_ids, mask, Wq, bq, Wk, bk, Wv, bv, avgs, std_devs):
    B, S, D = x.shape
    P = S - 1
    eps = jax.random.normal(jax.random.key(1234), (B, 2), dtype=jnp.float32)

    grid_spec = pltpu.PrefetchScalarGridSpec(
        num_scalar_prefetch=1,
        grid=(B,),
        in_specs=[
            pl.BlockSpec((1, S, D), lambda b, ids: (b, 0, 0)),
            pl.BlockSpec((D, D), lambda b, ids: (0, 0)),
            pl.BlockSpec((1, D), lambda b, ids: (0, 0)),
            pl.BlockSpec((D, D), lambda b, ids: (0, 0)),
            pl.BlockSpec((1, D), lambda b, ids: (0, 0)),
            pl.BlockSpec((D, D), lambda b, ids: (0, 0)),
            pl.BlockSpec((1, D), lambda b, ids: (0, 0)),
            pl.BlockSpec((1, 2, P), lambda b, ids: (ids[b], 0, 0)),
            pl.BlockSpec((1, 2, P), lambda b, ids: (ids[b], 0, 0)),
            pl.BlockSpec((1, 1, 2), lambda b, ids: (b, 0, 0)),
        ],
        out_specs=[
            pl.BlockSpec((1, S, D), lambda b, ids: (b, 0, 0)),
            pl.BlockSpec((1, 4, SPAD), lambda b, ids: (b, 0, 0)),
            pl.BlockSpec((1, 4, SPAD), lambda b, ids: (b, 0, 0)),
        ],
    )
    v, w, idx = pl.pallas_call(
        _score_body,
        grid_spec=grid_spec,
        out_shape=[
            jax.ShapeDtypeStruct((B, S, D), jnp.float32),
            jax.ShapeDtypeStruct((B, 4, SPAD), jnp.float32),
            jax.ShapeDtypeStruct((B, 4, SPAD), jnp.int32),
        ],
    )(img_ids, x, Wq, bq.reshape(1, D), Wk, bk.reshape(1, D), Wv,
      bv.reshape(1, D), avgs, std_devs, eps.reshape(B, 1, 2))

    wexp = jnp.broadcast_to(
        w.reshape(B, 4 * SPAD)[:, :, None], (B, 4 * SPAD, 16)
    ).reshape(B, 4 * SPAD * 16)
    sc_combine = _make_sc_combine(B, S, D)
    return sc_combine(v.reshape(B * S, D), idx.reshape(B, 4 * SPAD), wexp)
